# baseline (speedup 1.0000x reference)
import functools

import jax
import jax.numpy as jnp
from jax import lax
from jax.experimental import pallas as pl
from jax.experimental.pallas import tpu as pltpu
from jax.experimental.pallas import tpu_sc as plsc

N = 10000
NP = 10240
E = 320000
EB = 1280
NB = 640
K = 80
NEG = 0.2
ETE = 16
DOUTS = [128, 256, 192, 128, 64]
DINS = [128, 128, 256, 192, 128]
W_L = [64, 64, 96, 64, 32]
NCALLS = [1, 2, 1, 1, 1]

_f32 = jnp.float32
_HIGH = lax.Precision.HIGHEST


def _leaky(v):
    return jnp.where(v >= 0, v, NEG * v)


def _dot(a, b):
    return jnp.dot(a, b, precision=_HIGH, preferred_element_type=_f32)


def _prep_body(x4_ref, w4ea_ref, w4a_ref, w4ts_ref,
               eae_ref, s_ref, maxs_ref):
    @pl.when(pl.program_id(0) == 0)
    def _():
        maxs_ref[...] = jnp.full((8, 128), -jnp.inf, _f32)

    x4 = x4_ref[...]
    rows = lax.broadcasted_iota(jnp.int32, (8, 128), 0)
    upd = jnp.full((8, 128), -jnp.inf, _f32)

    def dotd(a, b):
        return jnp.dot(a, b, preferred_element_type=_f32)

    for l in range(5):
        eae4 = _leaky(dotd(x4, w4ea_ref[l]))
        eae_ref[l] = eae4
        s4 = dotd(eae4, w4a_ref[l]) + dotd(x4, w4ts_ref[l])
        s_ref[l] = s4
        upd = jnp.where(rows == l, jnp.max(s4), upd)
    maxs_ref[...] = jnp.maximum(maxs_ref[...], upd)


def _node_part(h, nt, hw_ref, hb_ref, wi_ref, wj_ref, wx_ref,
               ai_ref, aj_ref, hx_refs, mab_ref):
    dout = hw_ref.shape[2]
    h2 = jnp.zeros((h.shape[0], dout), _f32)
    for t in range(3):
        yt = _dot(h, hw_ref[t]) + hb_ref[t]
        h2 = h2 + jnp.where(nt == float(t), yt, 0.0)
    ai = _dot(h2, wi_ref[...])
    aj = _dot(h2, wj_ref[...])
    hx = _dot(h2, wx_ref[...])
    ai_ref[...] = ai
    aj_ref[...] = aj
    w = dout // len(hx_refs)
    for k, hr in enumerate(hx_refs):
        hr[...] = hx[:, k * w:(k + 1) * w]

    @pl.when(pl.program_id(0) == 0)
    def _():
        mab_ref[...] = jnp.full((8, 128), -jnp.inf, _f32)

    rows = lax.broadcasted_iota(jnp.int32, (8, 128), 0)
    upd = jnp.where(rows == 0, jnp.max(ai),
                    jnp.where(rows == 1, jnp.max(aj), -jnp.inf))
    mab_ref[...] = jnp.maximum(mab_ref[...], upd)


def _head_body(x_ref, nt_ref, hw_ref, hb_ref, wi_ref, wj_ref, wx_ref,
               ai_ref, aj_ref, *rest):
    _node_part(x_ref[...], nt_ref[...], hw_ref, hb_ref, wi_ref, wj_ref,
               wx_ref, ai_ref, aj_ref, list(rest[:-1]), rest[-1])


def _merge(gx_refs, g32a_ref, g32b_ref, we16_ref, lb_ref, lng_ref, lnb_ref,
           relu):
    gx = jnp.concatenate([r[...] for r in gx_refs], axis=1)
    g = g32a_ref[...] + g32b_ref[...]
    g16 = g[:, :16]
    s = g[:, 16:17]
    out = (gx + _dot(g16, we16_ref[...]) + s * lb_ref[...]) / (s + 1e-16)
    mu = jnp.mean(out, axis=1, keepdims=True)
    var = jnp.mean((out - mu) ** 2, axis=1, keepdims=True)
    h = (out - mu) / jnp.sqrt(var + 1e-5) * lng_ref[...] + lnb_ref[...]
    if relu:
        h = jnp.maximum(h, 0.0)
    return h


def _make_mid_body(ngx):
    def body(*refs):
        gx_refs = refs[:ngx]
        (g32a_ref, g32b_ref, nt_ref, we16_ref, lb_ref, lng_ref, lnb_ref,
         hw_ref, hb_ref, wi_ref, wj_ref, wx_ref, ai_ref, aj_ref) = \
            refs[ngx:ngx + 14]
        rest = refs[ngx + 14:]
        h = _merge(gx_refs, g32a_ref, g32b_ref, we16_ref, lb_ref, lng_ref,
                   lnb_ref, relu=True)
        _node_part(h, nt_ref[...], hw_ref, hb_ref, wi_ref, wj_ref, wx_ref,
                   ai_ref, aj_ref, list(rest[:-1]), rest[-1])
    return body


def _make_tail_body(ngx):
    def body(*refs):
        gx_refs = refs[:ngx]
        (g32a_ref, g32b_ref, we16_ref, lb_ref, lng_ref, lnb_ref,
         h_ref) = refs[ngx:]
        h_ref[...] = _merge(gx_refs, g32a_ref, g32b_ref, we16_ref, lb_ref,
                            lng_ref, lnb_ref, relu=False)
    return body


def _splat(v16, r):
    idx = jnp.full((16,), r, jnp.int32)
    return jnp.take_along_axis(v16, idx, axis=0, mode="promise_in_bounds")


EPC = E // 16
NCH = EPC // K
NBK = 10
NBLK = NCH // NBK


def _make_edge_kernel(w):
    nbk = 50 if w <= 64 else NBK
    nblk = NCH // nbk
    mesh = plsc.VectorSubcoreMesh(core_axis_name="c", subcore_axis_name="s")
    out_type = (
        jax.ShapeDtypeStruct((NP, w), _f32),
        jax.ShapeDtypeStruct((NP, w), _f32),
        jax.ShapeDtypeStruct((NP, 32), _f32),
        jax.ShapeDtypeStruct((NP, 32), _f32),
    )
    scratch = [
        pltpu.VMEM((NP,), _f32),
        pltpu.VMEM((NP,), _f32),
        pltpu.VMEM((nbk, K), jnp.int32),
        pltpu.VMEM((nbk, K), jnp.int32),
        pltpu.VMEM((nbk, K), jnp.int32),
        pltpu.VMEM((nbk, K), jnp.int32),
        pltpu.VMEM((nbk, K), _f32),
        pltpu.VMEM((nbk, K), _f32),
        pltpu.VMEM((K, ETE), _f32),
        pltpu.VMEM((K, ETE), _f32),
        pltpu.VMEM((K, w), _f32),
        pltpu.VMEM((K, w), _f32),
        pltpu.VMEM((K, 32), _f32),
        pltpu.VMEM((K, 32), _f32),
        pltpu.VMEM((16,), _f32),
        pltpu.VMEM_SHARED((NP, w), _f32),
        pltpu.VMEM_SHARED((NP, 32), _f32),
        pltpu.SemaphoreType.DMA,
        pltpu.SemaphoreType.DMA,
        pltpu.SemaphoreType.DMA,
        pltpu.SemaphoreType.DMA,
        pltpu.SemaphoreType.DMA,
        pltpu.SemaphoreType.DMA,
    ]

    def body(src_hbm, dst_hbm, se_hbm, eae_hbm, ai_hbm, aj_hbm,
             hxa_hbm, hxb_hbm, shift_hbm,
             gxa_out, gxb_out, g32a_out, g32b_out,
             ai_v, aj_v, srcb0_v, srcb1_v, dstb0_v, dstb1_v, eb0_v, eb1_v,
             eae0_v, eae1_v, rows0_v, rows1_v, r320_v, r321_v,
             shift_v, accw_sh, acc32_sh, sg0, sg1, ss0, ss1, sb0, sb1):
        c = lax.axis_index("c")
        sid = lax.axis_index("s")
        pltpu.sync_copy(ai_hbm, ai_v)
        pltpu.sync_copy(aj_hbm, aj_v)
        pltpu.sync_copy(shift_hbm, shift_v)

        sbufs = [(srcb0_v, dstb0_v, eb0_v, sb0),
                 (srcb1_v, dstb1_v, eb1_v, sb1)]
        bufs = [(rows0_v, r320_v, eae0_v, sg0, ss0),
                (rows1_v, r321_v, eae1_v, sg1, ss1)]
        z16 = jnp.zeros((16,), _f32)

        def zrow(rr, carry):
            for jc in range(w // 16):
                rows0_v[rr, pl.ds(jc * 16, 16)] = z16
            r320_v[rr, pl.ds(0, 16)] = z16
            r320_v[rr, pl.ds(16, 16)] = z16
            return carry

        lax.fori_loop(0, K, zrow, 0)
        stripe = NP // 16

        def zcp(k, carry):
            base = sid * stripe + k * K
            pltpu.sync_copy(rows0_v, accw_sh.at[pl.ds(base, K)])
            pltpu.sync_copy(r320_v, acc32_sh.at[pl.ds(base, K)])
            return carry

        lax.fori_loop(0, stripe // K, zcp, 0)
        plsc.subcore_barrier()

        shiftvec = shift_v[...]
        lanei = lax.iota(jnp.int32, 16)

        def fire_block(j, p):
            srcb, dstb, eb, sb = sbufs[p]
            sl = pl.ds(j * nbk, nbk)
            pltpu.async_copy(src_hbm.at[sid, sl], srcb, sb)
            pltpu.async_copy(dst_hbm.at[sid, sl], dstb, sb)
            pltpu.async_copy(se_hbm.at[sid, sl], eb, sb)

        def wait_block(j, p):
            srcb, dstb, eb, sb = sbufs[p]
            sl = pl.ds(j * nbk, nbk)
            pltpu.make_async_copy(src_hbm.at[sid, sl], srcb, sb).wait()
            pltpu.make_async_copy(dst_hbm.at[sid, sl], dstb, sb).wait()
            pltpu.make_async_copy(se_hbm.at[sid, sl], eb, sb).wait()

        def drain_scatter(b, dstb):
            rows_b, r32_b, eae_b, sg, ss = bufs[b]
            pltpu.make_async_copy(rows_b, accw_sh.at[dstb.at[0]],
                                  ss).wait()

            @pl.when(c == b)
            def _():
                pltpu.make_async_copy(r32_b, acc32_sh.at[dstb.at[0]],
                                      ss).wait()

        def process_block(j, p):
            srcb, dstb, eb, sb = sbufs[p]

            def epass(ii, carry):
                for g in range(K // 16):
                    sl = pl.ds(g * 16, 16)
                    zv = (plsc.load_gather(ai_v, [dstb[ii, sl]])
                          + plsc.load_gather(aj_v, [srcb[ii, sl]])
                          + eb[ii, sl])
                    zv = jnp.where(zv >= 0, zv, NEG * zv)
                    eb[ii, sl] = jnp.exp(zv - shiftvec)
                return carry

            lax.fori_loop(0, nbk, epass, 0)

            def sstep(s, carry):
                for b in (0, 1):
                    ii = 2 * s + b
                    i_glob = j * nbk + ii
                    rows_b, r32_b, eae_b, sg, ss = bufs[b]

                    @pl.when(i_glob >= 2)
                    def _():
                        drain_scatter(b, dstb)

                    @pl.when(c == 0)
                    def _():
                        pltpu.async_copy(hxa_hbm.at[srcb.at[ii]], rows_b,
                                         sg)

                    @pl.when(c == 1)
                    def _():
                        pltpu.async_copy(hxb_hbm.at[srcb.at[ii]], rows_b,
                                         sg)

                    @pl.when(c == b)
                    def _():
                        off = sid * EPC + i_glob * K
                        pltpu.async_copy(eae_hbm.at[pl.ds(off, K)], eae_b,
                                         sg)
                for b in (0, 1):
                    ii = 2 * s + b
                    i_glob = j * nbk + ii
                    rows_b, r32_b, eae_b, sg, ss = bufs[b]
                    pltpu.make_async_copy(hxa_hbm.at[srcb.at[ii]], rows_b,
                                          sg).wait()

                    @pl.when(c == b)
                    def _():
                        off = sid * EPC + i_glob * K
                        pltpu.make_async_copy(eae_hbm.at[pl.ds(off, K)],
                                              eae_b, sg).wait()

                    for g in range(K // 16):
                        e16 = eb[ii, pl.ds(g * 16, 16)]
                        for r in range(16):
                            row = g * 16 + r
                            spl = _splat(e16, r)
                            for jc in range(w // 16):
                                cs = pl.ds(jc * 16, 16)
                                rows_b[row, cs] = rows_b[row, cs] * spl

                    @pl.when(c == b)
                    def _():
                        for g in range(K // 16):
                            e16 = eb[ii, pl.ds(g * 16, 16)]
                            for r in range(16):
                                row = g * 16 + r
                                spl = _splat(e16, r)
                                r32_b[row, pl.ds(0, 16)] = \
                                    eae_b[row, pl.ds(0, 16)] * spl
                                r32_b[row, pl.ds(16, 16)] = \
                                    jnp.where(lanei == 0, spl, 0.0)

                    pltpu.async_copy(rows_b, accw_sh.at[dstb.at[ii]], ss,
                                     add=True)

                    @pl.when(c == b)
                    def _():
                        pltpu.async_copy(r32_b, acc32_sh.at[dstb.at[ii]],
                                         ss, add=True)
                return carry

            lax.fori_loop(0, nbk // 2, sstep, 0)

        fire_block(0, 0)

        def blockloop(j, carry):
            @pl.when(j % 2 == 0)
            def _():
                wait_block(j, 0)

                @pl.when(j + 1 < nblk)
                def _():
                    fire_block(j + 1, 1)

                process_block(j, 0)

            @pl.when(j % 2 == 1)
            def _():
                wait_block(j, 1)

                @pl.when(j + 1 < nblk)
                def _():
                    fire_block(j + 1, 0)

                process_block(j, 1)

            return carry

        lax.fori_loop(0, nblk, blockloop, 0)
        drain_scatter(0, dstb0_v)
        drain_scatter(1, dstb0_v)
        plsc.subcore_barrier()
        base = sid * stripe

        @pl.when(c == 0)
        def _():
            pltpu.sync_copy(accw_sh.at[pl.ds(base, stripe)],
                            gxa_out.at[pl.ds(base, stripe)])
            pltpu.sync_copy(acc32_sh.at[pl.ds(base, stripe)],
                            g32a_out.at[pl.ds(base, stripe)])

        @pl.when(c == 1)
        def _():
            pltpu.sync_copy(accw_sh.at[pl.ds(base, stripe)],
                            gxb_out.at[pl.ds(base, stripe)])
            pltpu.sync_copy(acc32_sh.at[pl.ds(base, stripe)],
                            g32b_out.at[pl.ds(base, stripe)])

    return pl.kernel(body, out_type=out_type, mesh=mesh,
                     scratch_types=scratch,
                     compiler_params=pltpu.CompilerParams(
                         needs_layout_passes=False,
                         use_tc_tiling_on_sc=False))


def _full(shape):
    return pl.BlockSpec(shape, lambda i: tuple(0 for _ in shape))


def kernel(x, edge_index, node_type, edge_type, edge_attr, params):
    src = edge_index[0]
    dst = edge_index[1]
    xp = jnp.pad(x, ((0, NP - N), (0, 0)))
    ntf = jnp.pad(node_type.astype(_f32), (0, NP - N)).reshape(NP, 1)
    ps = [params['conv%d' % (i + 1)] for i in range(5)]

    oh = jax.nn.one_hot(edge_type, 6, dtype=_f32)
    x4 = jnp.concatenate([edge_attr, oh, jnp.zeros((E, 10), _f32)],
                         axis=1).reshape(E // 4, 128)
    w4ea_l, w4a_l, w4ts_l = [], [], []
    for l in range(5):
        p = ps[l]
        d = DOUTS[l]
        aw = p['att_W'][:, 0]
        we_v = aw[2 * d:2 * d + ETE]
        wa_v = aw[2 * d + ETE:2 * d + 2 * ETE]
        emb = p['edge_type_emb']
        ts = jnp.where(emb >= 0, emb, NEG * emb) @ we_v
        w4ea = jnp.zeros((128, 64), _f32)
        w4a = jnp.zeros((64, 4), _f32)
        w4ts = jnp.zeros((128, 4), _f32)
        for m in range(4):
            w4ea = w4ea.at[32 * m:32 * m + 16,
                           16 * m:16 * m + 16].set(p['edge_attr_W'])
            w4a = w4a.at[16 * m:16 * m + 16, m].set(wa_v)
            w4ts = w4ts.at[32 * m + 16:32 * m + 22, m].set(ts)
        w4ea_l.append(w4ea)
        w4a_l.append(w4a)
        w4ts_l.append(w4ts)
    w4ea_all = jnp.stack(w4ea_l)
    w4a_all = jnp.stack(w4a_l)
    w4ts_all = jnp.stack(w4ts_l)

    EB4 = 1600
    eae_p, s_p, maxs = pl.pallas_call(
        _prep_body,
        grid=(E // 4 // EB4,),
        in_specs=[
            pl.BlockSpec((EB4, 128), lambda i: (i, 0)),
            _full((5, 128, 64)),
            _full((5, 64, 4)),
            _full((5, 128, 4)),
        ],
        out_specs=[
            pl.BlockSpec((5, EB4, 64), lambda i: (0, i, 0)),
            pl.BlockSpec((5, EB4, 4), lambda i: (0, i, 0)),
            _full((8, 128)),
        ],
        out_shape=[
            jax.ShapeDtypeStruct((5, E // 4, 64), _f32),
            jax.ShapeDtypeStruct((5, E // 4, 4), _f32),
            jax.ShapeDtypeStruct((8, 128), _f32),
        ],
    )(x4, w4ea_all, w4a_all, w4ts_all)
    eae_all = eae_p.reshape(5, E, 16)
    s_all = s_p.reshape(5, E)

    def node_weights(l):
        p = ps[l]
        d = DOUTS[l]
        return (p['hetero_W'], p['hetero_b'][:, None, :],
                p['att_W'][:d], p['att_W'][d:2 * d], p['lin_W'][:d])

    def node_specs(l):
        din, d = DINS[l], DOUTS[l]
        w = W_L[l]
        nh = 2 * NCALLS[l]
        ins = [_full((3, din, d)), _full((3, 1, d)), _full((d, 1)),
               _full((d, 1)), _full((d, d))]
        outs = ([pl.BlockSpec((NB, 1), lambda i: (i, 0))] * 2
                + [pl.BlockSpec((NB, w), lambda i: (i, 0))] * nh
                + [_full((8, 128))])
        oshape = ([jax.ShapeDtypeStruct((NP, 1), _f32)] * 2
                  + [jax.ShapeDtypeStruct((NP, w), _f32)] * nh
                  + [jax.ShapeDtypeStruct((8, 128), _f32)])
        return ins, outs, oshape

    ins0, outs0, oshape0 = node_specs(0)
    ai, aj, *hx_list, mab = pl.pallas_call(
        _head_body,
        grid=(NP // NB,),
        in_specs=[pl.BlockSpec((NB, DINS[0]), lambda i: (i, 0)),
                  pl.BlockSpec((NB, 1), lambda i: (i, 0))] + ins0,
        out_specs=outs0,
        out_shape=oshape0,
    )(xp, ntf, *node_weights(0))

    src3 = src.reshape(16, NCH, K)
    dst3 = dst.reshape(16, NCH, K)
    s4 = s_all.reshape(5, 16, NCH, K)

    h = None
    for l in range(5):
        d = DOUTS[l]
        w = W_L[l]
        nc = NCALLS[l]
        shift = jnp.maximum(mab[0, 0] + mab[1, 0] + maxs[l, 0], 0.0)
        shift_arr = jnp.full((16,), shift, _f32)
        edge_k = _make_edge_kernel(w)
        gx_list = []
        g32a = g32b = None
        for k in range(nc):
            gxa, gxb, g32ak, g32bk = edge_k(src3, dst3, s4[l], eae_all[l],
                                            ai.reshape(NP), aj.reshape(NP),
                                            hx_list[2 * k],
                                            hx_list[2 * k + 1],
                                            shift_arr)
            gx_list += [gxa, gxb]
            if k == 0:
                g32a, g32b = g32ak, g32bk
        p = ps[l]
        merge_w = (p['lin_W'][d:], p['lin_b'][None, :],
                   p['ln_g'][None, :], p['ln_b'][None, :])
        merge_specs = [_full((16, d)), _full((1, d)), _full((1, d)),
                       _full((1, d))]
        gspecs = ([pl.BlockSpec((NB, w), lambda i: (i, 0))] * (2 * nc)
                  + [pl.BlockSpec((NB, 32), lambda i: (i, 0))] * 2)
        if l < 4:
            insn, outsn, oshapen = node_specs(l + 1)
            ai, aj, *hx_list, mab = pl.pallas_call(
                _make_mid_body(2 * nc),
                grid=(NP // NB,),
                in_specs=gspecs
                + [pl.BlockSpec((NB, 1), lambda i: (i, 0))]
                + merge_specs + insn,
                out_specs=outsn,
                out_shape=oshapen,
            )(*gx_list, g32a, g32b, ntf, *merge_w, *node_weights(l + 1))
        else:
            h = pl.pallas_call(
                _make_tail_body(2 * nc),
                grid=(NP // NB,),
                in_specs=gspecs + merge_specs,
                out_specs=pl.BlockSpec((NB, d), lambda i: (i, 0)),
                out_shape=jax.ShapeDtypeStruct((NP, d), _f32),
            )(*gx_list, g32a, g32b, *merge_w)
    return h[:N]


# speedup vs baseline: 1.0915x; 1.0915x over previous
import functools

import jax
import jax.numpy as jnp
from jax import lax
from jax.experimental import pallas as pl
from jax.experimental.pallas import tpu as pltpu
from jax.experimental.pallas import tpu_sc as plsc

N = 10000
NP = 10240
E = 320000
EB = 1280
NB = 1024
K = 80
NEG = 0.2
ETE = 16
DOUTS = [128, 256, 192, 128, 64]
DINS = [128, 128, 256, 192, 128]
W_L = [64, 64, 96, 64, 32]
NCALLS = [1, 2, 1, 1, 1]

_f32 = jnp.float32
_HIGH = lax.Precision.HIGHEST


def _leaky(v):
    return jnp.where(v >= 0, v, NEG * v)


def _dot(a, b):
    return jnp.dot(a, b, preferred_element_type=_f32)


def _prep_body(x4_ref, w4ea_ref, w4a_ref, w4ts_ref,
               eae_ref, s_ref, maxs_ref):
    @pl.when(pl.program_id(0) == 0)
    def _():
        maxs_ref[...] = jnp.full((8, 128), -jnp.inf, _f32)

    x4 = x4_ref[...]
    rows = lax.broadcasted_iota(jnp.int32, (8, 128), 0)
    upd = jnp.full((8, 128), -jnp.inf, _f32)

    def dotd(a, b):
        return jnp.dot(a, b, preferred_element_type=_f32)

    for l in range(5):
        eae4 = _leaky(dotd(x4, w4ea_ref[l]))
        eae_ref[l] = eae4
        s4 = dotd(eae4, w4a_ref[l]) + dotd(x4, w4ts_ref[l])
        s_ref[l] = s4
        upd = jnp.where(rows == l, jnp.max(s4), upd)
    maxs_ref[...] = jnp.maximum(maxs_ref[...], upd)


def _node_part(h, nt, hw_ref, hb_ref, wi_ref, wj_ref, wx_ref,
               ai_ref, aj_ref, hx_refs, mab_ref):
    dout = hw_ref.shape[2]
    h2 = jnp.zeros((h.shape[0], dout), _f32)
    for t in range(3):
        yt = _dot(h, hw_ref[t]) + hb_ref[t]
        h2 = h2 + jnp.where(nt == float(t), yt, 0.0)
    ai = _dot(h2, wi_ref[...])
    aj = _dot(h2, wj_ref[...])
    hx = _dot(h2, wx_ref[...])
    ai_ref[...] = ai
    aj_ref[...] = aj
    w = dout // len(hx_refs)
    for k, hr in enumerate(hx_refs):
        hr[...] = hx[:, k * w:(k + 1) * w]

    @pl.when(pl.program_id(0) == 0)
    def _():
        mab_ref[...] = jnp.full((8, 128), -jnp.inf, _f32)

    rows = lax.broadcasted_iota(jnp.int32, (8, 128), 0)
    upd = jnp.where(rows == 0, jnp.max(ai),
                    jnp.where(rows == 1, jnp.max(aj), -jnp.inf))
    mab_ref[...] = jnp.maximum(mab_ref[...], upd)


def _head_body(x_ref, nt_ref, hw_ref, hb_ref, wi_ref, wj_ref, wx_ref,
               ai_ref, aj_ref, *rest):
    _node_part(x_ref[...], nt_ref[...], hw_ref, hb_ref, wi_ref, wj_ref,
               wx_ref, ai_ref, aj_ref, list(rest[:-1]), rest[-1])


def _merge(gx_refs, g32a_ref, g32b_ref, we16_ref, lb_ref, lng_ref, lnb_ref,
           relu):
    gx = jnp.concatenate([r[...] for r in gx_refs], axis=1)
    g = g32a_ref[...] + g32b_ref[...]
    g16 = g[:, :16]
    s = g[:, 16:17]
    out = (gx + _dot(g16, we16_ref[...]) + s * lb_ref[...]) / (s + 1e-16)
    mu = jnp.mean(out, axis=1, keepdims=True)
    var = jnp.mean((out - mu) ** 2, axis=1, keepdims=True)
    h = (out - mu) / jnp.sqrt(var + 1e-5) * lng_ref[...] + lnb_ref[...]
    if relu:
        h = jnp.maximum(h, 0.0)
    return h


def _make_mid_body(ngx):
    def body(*refs):
        gx_refs = refs[:ngx]
        (g32a_ref, g32b_ref, nt_ref, we16_ref, lb_ref, lng_ref, lnb_ref,
         hw_ref, hb_ref, wi_ref, wj_ref, wx_ref, ai_ref, aj_ref) = \
            refs[ngx:ngx + 14]
        rest = refs[ngx + 14:]
        h = _merge(gx_refs, g32a_ref, g32b_ref, we16_ref, lb_ref, lng_ref,
                   lnb_ref, relu=True)
        _node_part(h, nt_ref[...], hw_ref, hb_ref, wi_ref, wj_ref, wx_ref,
                   ai_ref, aj_ref, list(rest[:-1]), rest[-1])
    return body


def _make_tail_body(ngx):
    def body(*refs):
        gx_refs = refs[:ngx]
        (g32a_ref, g32b_ref, we16_ref, lb_ref, lng_ref, lnb_ref,
         h_ref) = refs[ngx:]
        h_ref[...] = _merge(gx_refs, g32a_ref, g32b_ref, we16_ref, lb_ref,
                            lng_ref, lnb_ref, relu=False)
    return body


def _splat(v16, r):
    idx = jnp.full((16,), r, jnp.int32)
    return jnp.take_along_axis(v16, idx, axis=0, mode="promise_in_bounds")


EPC = E // 16
NCH = EPC // K
NBK = 10
NBLK = NCH // NBK


def _make_edge_kernel(w):
    nbk = 50 if w <= 64 else NBK
    nblk = NCH // nbk
    mesh = plsc.VectorSubcoreMesh(core_axis_name="c", subcore_axis_name="s")
    out_type = (
        jax.ShapeDtypeStruct((NP, w), _f32),
        jax.ShapeDtypeStruct((NP, w), _f32),
        jax.ShapeDtypeStruct((NP, 32), _f32),
        jax.ShapeDtypeStruct((NP, 32), _f32),
    )
    scratch = [
        pltpu.VMEM((NP,), _f32),
        pltpu.VMEM((NP,), _f32),
        pltpu.VMEM((nbk, K), jnp.int32),
        pltpu.VMEM((nbk, K), jnp.int32),
        pltpu.VMEM((nbk, K), jnp.int32),
        pltpu.VMEM((nbk, K), jnp.int32),
        pltpu.VMEM((nbk, K), _f32),
        pltpu.VMEM((nbk, K), _f32),
        pltpu.VMEM((K, ETE), _f32),
        pltpu.VMEM((K, ETE), _f32),
        pltpu.VMEM((K, w), _f32),
        pltpu.VMEM((K, w), _f32),
        pltpu.VMEM((K, 32), _f32),
        pltpu.VMEM((K, 32), _f32),
        pltpu.VMEM((16,), _f32),
        pltpu.VMEM_SHARED((NP, w), _f32),
        pltpu.VMEM_SHARED((NP, 32), _f32),
        pltpu.SemaphoreType.DMA,
        pltpu.SemaphoreType.DMA,
        pltpu.SemaphoreType.DMA,
        pltpu.SemaphoreType.DMA,
        pltpu.SemaphoreType.DMA,
        pltpu.SemaphoreType.DMA,
    ]

    def body(src_hbm, dst_hbm, se_hbm, eae_hbm, ai_hbm, aj_hbm,
             hxa_hbm, hxb_hbm, shift_hbm,
             gxa_out, gxb_out, g32a_out, g32b_out,
             ai_v, aj_v, srcb0_v, srcb1_v, dstb0_v, dstb1_v, eb0_v, eb1_v,
             eae0_v, eae1_v, rows0_v, rows1_v, r320_v, r321_v,
             shift_v, accw_sh, acc32_sh, sg0, sg1, ss0, ss1, sb0, sb1):
        c = lax.axis_index("c")
        sid = lax.axis_index("s")
        pltpu.sync_copy(ai_hbm, ai_v)
        pltpu.sync_copy(aj_hbm, aj_v)
        pltpu.sync_copy(shift_hbm, shift_v)

        sbufs = [(srcb0_v, dstb0_v, eb0_v, sb0),
                 (srcb1_v, dstb1_v, eb1_v, sb1)]
        bufs = [(rows0_v, r320_v, eae0_v, sg0, ss0),
                (rows1_v, r321_v, eae1_v, sg1, ss1)]
        z16 = jnp.zeros((16,), _f32)

        def zrow(rr, carry):
            for jc in range(w // 16):
                rows0_v[rr, pl.ds(jc * 16, 16)] = z16
            r320_v[rr, pl.ds(0, 16)] = z16
            r320_v[rr, pl.ds(16, 16)] = z16
            return carry

        lax.fori_loop(0, K, zrow, 0)
        stripe = NP // 16

        def zcp(k, carry):
            base = sid * stripe + k * K
            pltpu.sync_copy(rows0_v, accw_sh.at[pl.ds(base, K)])
            pltpu.sync_copy(r320_v, acc32_sh.at[pl.ds(base, K)])
            return carry

        lax.fori_loop(0, stripe // K, zcp, 0)
        plsc.subcore_barrier()

        shiftvec = shift_v[...]
        lanei = lax.iota(jnp.int32, 16)

        def fire_block(j, p):
            srcb, dstb, eb, sb = sbufs[p]
            sl = pl.ds(j * nbk, nbk)
            pltpu.async_copy(src_hbm.at[sid, sl], srcb, sb)
            pltpu.async_copy(dst_hbm.at[sid, sl], dstb, sb)
            pltpu.async_copy(se_hbm.at[sid, sl], eb, sb)

        def wait_block(j, p):
            srcb, dstb, eb, sb = sbufs[p]
            sl = pl.ds(j * nbk, nbk)
            pltpu.make_async_copy(src_hbm.at[sid, sl], srcb, sb).wait()
            pltpu.make_async_copy(dst_hbm.at[sid, sl], dstb, sb).wait()
            pltpu.make_async_copy(se_hbm.at[sid, sl], eb, sb).wait()

        def drain_scatter(b, dstb):
            rows_b, r32_b, eae_b, sg, ss = bufs[b]
            pltpu.make_async_copy(rows_b, accw_sh.at[dstb.at[0]],
                                  ss).wait()

            @pl.when(c == b)
            def _():
                pltpu.make_async_copy(r32_b, acc32_sh.at[dstb.at[0]],
                                      ss).wait()

        def process_block(j, p):
            srcb, dstb, eb, sb = sbufs[p]

            def epass(ii, carry):
                for g in range(K // 16):
                    sl = pl.ds(g * 16, 16)
                    zv = (plsc.load_gather(ai_v, [dstb[ii, sl]])
                          + plsc.load_gather(aj_v, [srcb[ii, sl]])
                          + eb[ii, sl])
                    zv = jnp.where(zv >= 0, zv, NEG * zv)
                    eb[ii, sl] = jnp.exp(zv - shiftvec)
                return carry

            lax.fori_loop(0, nbk, epass, 0)

            def sstep(s, carry):
                for b in (0, 1):
                    ii = 2 * s + b
                    i_glob = j * nbk + ii
                    rows_b, r32_b, eae_b, sg, ss = bufs[b]

                    @pl.when(i_glob >= 2)
                    def _():
                        drain_scatter(b, dstb)

                    @pl.when(c == 0)
                    def _():
                        pltpu.async_copy(hxa_hbm.at[srcb.at[ii]], rows_b,
                                         sg)

                    @pl.when(c == 1)
                    def _():
                        pltpu.async_copy(hxb_hbm.at[srcb.at[ii]], rows_b,
                                         sg)

                    @pl.when(c == b)
                    def _():
                        off = sid * EPC + i_glob * K
                        pltpu.async_copy(eae_hbm.at[pl.ds(off, K)], eae_b,
                                         sg)
                for b in (0, 1):
                    ii = 2 * s + b
                    i_glob = j * nbk + ii
                    rows_b, r32_b, eae_b, sg, ss = bufs[b]
                    pltpu.make_async_copy(hxa_hbm.at[srcb.at[ii]], rows_b,
                                          sg).wait()

                    @pl.when(c == b)
                    def _():
                        off = sid * EPC + i_glob * K
                        pltpu.make_async_copy(eae_hbm.at[pl.ds(off, K)],
                                              eae_b, sg).wait()

                    for g in range(K // 16):
                        e16 = eb[ii, pl.ds(g * 16, 16)]
                        for r in range(16):
                            row = g * 16 + r
                            spl = _splat(e16, r)
                            for jc in range(w // 16):
                                cs = pl.ds(jc * 16, 16)
                                rows_b[row, cs] = rows_b[row, cs] * spl

                    @pl.when(c == b)
                    def _():
                        for g in range(K // 16):
                            e16 = eb[ii, pl.ds(g * 16, 16)]
                            for r in range(16):
                                row = g * 16 + r
                                spl = _splat(e16, r)
                                r32_b[row, pl.ds(0, 16)] = \
                                    eae_b[row, pl.ds(0, 16)] * spl
                                r32_b[row, pl.ds(16, 16)] = \
                                    jnp.where(lanei == 0, spl, 0.0)

                    pltpu.async_copy(rows_b, accw_sh.at[dstb.at[ii]], ss,
                                     add=True)

                    @pl.when(c == b)
                    def _():
                        pltpu.async_copy(r32_b, acc32_sh.at[dstb.at[ii]],
                                         ss, add=True)
                return carry

            lax.fori_loop(0, nbk // 2, sstep, 0)

        fire_block(0, 0)

        def blockloop(j, carry):
            @pl.when(j % 2 == 0)
            def _():
                wait_block(j, 0)

                @pl.when(j + 1 < nblk)
                def _():
                    fire_block(j + 1, 1)

                process_block(j, 0)

            @pl.when(j % 2 == 1)
            def _():
                wait_block(j, 1)

                @pl.when(j + 1 < nblk)
                def _():
                    fire_block(j + 1, 0)

                process_block(j, 1)

            return carry

        lax.fori_loop(0, nblk, blockloop, 0)
        drain_scatter(0, dstb0_v)
        drain_scatter(1, dstb0_v)
        plsc.subcore_barrier()
        base = sid * stripe

        @pl.when(c == 0)
        def _():
            pltpu.sync_copy(accw_sh.at[pl.ds(base, stripe)],
                            gxa_out.at[pl.ds(base, stripe)])
            pltpu.sync_copy(acc32_sh.at[pl.ds(base, stripe)],
                            g32a_out.at[pl.ds(base, stripe)])

        @pl.when(c == 1)
        def _():
            pltpu.sync_copy(accw_sh.at[pl.ds(base, stripe)],
                            gxb_out.at[pl.ds(base, stripe)])
            pltpu.sync_copy(acc32_sh.at[pl.ds(base, stripe)],
                            g32b_out.at[pl.ds(base, stripe)])

    return pl.kernel(body, out_type=out_type, mesh=mesh,
                     scratch_types=scratch,
                     compiler_params=pltpu.CompilerParams(
                         needs_layout_passes=False,
                         use_tc_tiling_on_sc=False))


def _full(shape):
    return pl.BlockSpec(shape, lambda i: tuple(0 for _ in shape))


def kernel(x, edge_index, node_type, edge_type, edge_attr, params):
    src = edge_index[0]
    dst = edge_index[1]
    xp = jnp.pad(x, ((0, NP - N), (0, 0)))
    ntf = jnp.pad(node_type.astype(_f32), (0, NP - N)).reshape(NP, 1)
    ps = [params['conv%d' % (i + 1)] for i in range(5)]

    oh = jax.nn.one_hot(edge_type, 6, dtype=_f32)
    x4 = jnp.concatenate([edge_attr, oh, jnp.zeros((E, 10), _f32)],
                         axis=1).reshape(E // 4, 128)
    w4ea_l, w4a_l, w4ts_l = [], [], []
    for l in range(5):
        p = ps[l]
        d = DOUTS[l]
        aw = p['att_W'][:, 0]
        we_v = aw[2 * d:2 * d + ETE]
        wa_v = aw[2 * d + ETE:2 * d + 2 * ETE]
        emb = p['edge_type_emb']
        ts = jnp.where(emb >= 0, emb, NEG * emb) @ we_v
        w4ea = jnp.zeros((128, 64), _f32)
        w4a = jnp.zeros((64, 4), _f32)
        w4ts = jnp.zeros((128, 4), _f32)
        for m in range(4):
            w4ea = w4ea.at[32 * m:32 * m + 16,
                           16 * m:16 * m + 16].set(p['edge_attr_W'])
            w4a = w4a.at[16 * m:16 * m + 16, m].set(wa_v)
            w4ts = w4ts.at[32 * m + 16:32 * m + 22, m].set(ts)
        w4ea_l.append(w4ea)
        w4a_l.append(w4a)
        w4ts_l.append(w4ts)
    w4ea_all = jnp.stack(w4ea_l)
    w4a_all = jnp.stack(w4a_l)
    w4ts_all = jnp.stack(w4ts_l)

    EB4 = 1600
    eae_p, s_p, maxs = pl.pallas_call(
        _prep_body,
        grid=(E // 4 // EB4,),
        in_specs=[
            pl.BlockSpec((EB4, 128), lambda i: (i, 0)),
            _full((5, 128, 64)),
            _full((5, 64, 4)),
            _full((5, 128, 4)),
        ],
        out_specs=[
            pl.BlockSpec((5, EB4, 64), lambda i: (0, i, 0)),
            pl.BlockSpec((5, EB4, 4), lambda i: (0, i, 0)),
            _full((8, 128)),
        ],
        out_shape=[
            jax.ShapeDtypeStruct((5, E // 4, 64), _f32),
            jax.ShapeDtypeStruct((5, E // 4, 4), _f32),
            jax.ShapeDtypeStruct((8, 128), _f32),
        ],
    )(x4, w4ea_all, w4a_all, w4ts_all)
    eae_all = eae_p.reshape(5, E, 16)
    s_all = s_p.reshape(5, E)

    def node_weights(l):
        p = ps[l]
        d = DOUTS[l]
        return (p['hetero_W'], p['hetero_b'][:, None, :],
                p['att_W'][:d], p['att_W'][d:2 * d], p['lin_W'][:d])

    def node_specs(l):
        din, d = DINS[l], DOUTS[l]
        w = W_L[l]
        nh = 2 * NCALLS[l]
        ins = [_full((3, din, d)), _full((3, 1, d)), _full((d, 1)),
               _full((d, 1)), _full((d, d))]
        outs = ([pl.BlockSpec((NB, 1), lambda i: (i, 0))] * 2
                + [pl.BlockSpec((NB, w), lambda i: (i, 0))] * nh
                + [_full((8, 128))])
        oshape = ([jax.ShapeDtypeStruct((NP, 1), _f32)] * 2
                  + [jax.ShapeDtypeStruct((NP, w), _f32)] * nh
                  + [jax.ShapeDtypeStruct((8, 128), _f32)])
        return ins, outs, oshape

    ins0, outs0, oshape0 = node_specs(0)
    ai, aj, *hx_list, mab = pl.pallas_call(
        _head_body,
        grid=(NP // NB,),
        in_specs=[pl.BlockSpec((NB, DINS[0]), lambda i: (i, 0)),
                  pl.BlockSpec((NB, 1), lambda i: (i, 0))] + ins0,
        out_specs=outs0,
        out_shape=oshape0,
    )(xp, ntf, *node_weights(0))

    src3 = src.reshape(16, NCH, K)
    dst3 = dst.reshape(16, NCH, K)
    s4 = s_all.reshape(5, 16, NCH, K)

    h = None
    for l in range(5):
        d = DOUTS[l]
        w = W_L[l]
        nc = NCALLS[l]
        shift = jnp.maximum(mab[0, 0] + mab[1, 0] + maxs[l, 0], 0.0)
        shift_arr = jnp.full((16,), shift, _f32)
        edge_k = _make_edge_kernel(w)
        gx_list = []
        g32a = g32b = None
        for k in range(nc):
            gxa, gxb, g32ak, g32bk = edge_k(src3, dst3, s4[l], eae_all[l],
                                            ai.reshape(NP), aj.reshape(NP),
                                            hx_list[2 * k],
                                            hx_list[2 * k + 1],
                                            shift_arr)
            gx_list += [gxa, gxb]
            if k == 0:
                g32a, g32b = g32ak, g32bk
        p = ps[l]
        merge_w = (p['lin_W'][d:], p['lin_b'][None, :],
                   p['ln_g'][None, :], p['ln_b'][None, :])
        merge_specs = [_full((16, d)), _full((1, d)), _full((1, d)),
                       _full((1, d))]
        gspecs = ([pl.BlockSpec((NB, w), lambda i: (i, 0))] * (2 * nc)
                  + [pl.BlockSpec((NB, 32), lambda i: (i, 0))] * 2)
        if l < 4:
            insn, outsn, oshapen = node_specs(l + 1)
            ai, aj, *hx_list, mab = pl.pallas_call(
                _make_mid_body(2 * nc),
                grid=(NP // NB,),
                in_specs=gspecs
                + [pl.BlockSpec((NB, 1), lambda i: (i, 0))]
                + merge_specs + insn,
                out_specs=outsn,
                out_shape=oshapen,
            )(*gx_list, g32a, g32b, ntf, *merge_w, *node_weights(l + 1))
        else:
            h = pl.pallas_call(
                _make_tail_body(2 * nc),
                grid=(NP // NB,),
                in_specs=gspecs + merge_specs,
                out_specs=pl.BlockSpec((NB, d), lambda i: (i, 0)),
                out_shape=jax.ShapeDtypeStruct((NP, d), _f32),
            )(*gx_list, g32a, g32b, *merge_w)
    return h[:N]


# speedup vs baseline: 1.0915x; 1.0000x over previous
import jax
import jax.numpy as jnp
from jax import lax
from jax.experimental import pallas as pl
from jax.experimental.pallas import tpu as pltpu
from jax.experimental.pallas import tpu_sc as plsc

N = 10000
NP = 10240
E = 320000
NB = 1024
K = 80
NEG = 0.2
ETE = 16
DOUTS = [128, 256, 192, 128, 64]
DINS = [128, 128, 256, 192, 128]
W_L = [64, 64, 96, 64, 32]
NCALLS = [1, 2, 1, 1, 1]

_f32 = jnp.float32


def _leaky(v):
    return jnp.where(v >= 0, v, NEG * v)


def _dot(a, b):
    return jnp.dot(a, b, preferred_element_type=_f32)


def _prep_body(x4_ref, w4ea_ref, w4a_ref, w4ts_ref,
               eae_ref, s_ref, maxs_ref):
    @pl.when(pl.program_id(0) == 0)
    def _():
        maxs_ref[...] = jnp.full((8, 128), -jnp.inf, _f32)

    x4 = x4_ref[...]
    rows = lax.broadcasted_iota(jnp.int32, (8, 128), 0)
    upd = jnp.full((8, 128), -jnp.inf, _f32)

    def dotd(a, b):
        return jnp.dot(a, b, preferred_element_type=_f32)

    for l in range(5):
        eae4 = _leaky(dotd(x4, w4ea_ref[l]))
        eae_ref[l] = eae4
        s4 = dotd(eae4, w4a_ref[l]) + dotd(x4, w4ts_ref[l])
        s_ref[l] = s4
        upd = jnp.where(rows == l, jnp.max(s4), upd)
    maxs_ref[...] = jnp.maximum(maxs_ref[...], upd)


def _node_part(h, nt, hw_ref, hb_ref, wi_ref, wj_ref, wx_ref,
               ai_ref, aj_ref, hx_refs, mab_ref):
    dout = hw_ref.shape[2]
    h2 = jnp.zeros((h.shape[0], dout), _f32)
    for t in range(3):
        yt = _dot(h, hw_ref[t]) + hb_ref[t]
        h2 = h2 + jnp.where(nt == float(t), yt, 0.0)
    ai = _dot(h2, wi_ref[...])
    aj = _dot(h2, wj_ref[...])
    hx = _dot(h2, wx_ref[...])
    ai_ref[...] = ai
    aj_ref[...] = aj
    w = dout // len(hx_refs)
    for k, hr in enumerate(hx_refs):
        hr[...] = hx[:, k * w:(k + 1) * w]

    @pl.when(pl.program_id(0) == 0)
    def _():
        mab_ref[...] = jnp.full((8, 128), -jnp.inf, _f32)

    rows = lax.broadcasted_iota(jnp.int32, (8, 128), 0)
    upd = jnp.where(rows == 0, jnp.max(ai),
                    jnp.where(rows == 1, jnp.max(aj), -jnp.inf))
    mab_ref[...] = jnp.maximum(mab_ref[...], upd)


def _head_body(x_ref, nt_ref, hw_ref, hb_ref, wi_ref, wj_ref, wx_ref,
               ai_ref, aj_ref, *rest):
    _node_part(x_ref[...], nt_ref[...], hw_ref, hb_ref, wi_ref, wj_ref,
               wx_ref, ai_ref, aj_ref, list(rest[:-1]), rest[-1])


def _merge(gx_refs, g32a_ref, g32b_ref, we16_ref, lb_ref, lng_ref, lnb_ref,
           relu):
    gx = jnp.concatenate([r[...] for r in gx_refs], axis=1)
    g = g32a_ref[...] + g32b_ref[...]
    g16 = g[:, :16]
    s = g[:, 16:17]
    out = (gx + _dot(g16, we16_ref[...]) + s * lb_ref[...]) / (s + 1e-16)
    mu = jnp.mean(out, axis=1, keepdims=True)
    var = jnp.mean((out - mu) ** 2, axis=1, keepdims=True)
    h = (out - mu) / jnp.sqrt(var + 1e-5) * lng_ref[...] + lnb_ref[...]
    if relu:
        h = jnp.maximum(h, 0.0)
    return h


def _make_mid_body(ngx):
    def body(*refs):
        gx_refs = refs[:ngx]
        (g32a_ref, g32b_ref, nt_ref, we16_ref, lb_ref, lng_ref, lnb_ref,
         hw_ref, hb_ref, wi_ref, wj_ref, wx_ref, ai_ref, aj_ref) = \
            refs[ngx:ngx + 14]
        rest = refs[ngx + 14:]
        h = _merge(gx_refs, g32a_ref, g32b_ref, we16_ref, lb_ref, lng_ref,
                   lnb_ref, relu=True)
        _node_part(h, nt_ref[...], hw_ref, hb_ref, wi_ref, wj_ref, wx_ref,
                   ai_ref, aj_ref, list(rest[:-1]), rest[-1])
    return body


def _make_tail_body(ngx):
    def body(*refs):
        gx_refs = refs[:ngx]
        (g32a_ref, g32b_ref, we16_ref, lb_ref, lng_ref, lnb_ref,
         h_ref) = refs[ngx:]
        h_ref[...] = _merge(gx_refs, g32a_ref, g32b_ref, we16_ref, lb_ref,
                            lng_ref, lnb_ref, relu=False)
    return body


def _splat(v16, r):
    idx = jnp.full((16,), r, jnp.int32)
    return jnp.take_along_axis(v16, idx, axis=0, mode="promise_in_bounds")


EPC = E // 16
NCH = EPC // K
NBK = 10
NBLK = NCH // NBK


def _make_edge_kernel(w):
    nbk = 50 if w <= 64 else NBK
    nblk = NCH // nbk
    mesh = plsc.VectorSubcoreMesh(core_axis_name="c", subcore_axis_name="s")
    out_type = (
        jax.ShapeDtypeStruct((NP, w), _f32),
        jax.ShapeDtypeStruct((NP, w), _f32),
        jax.ShapeDtypeStruct((NP, 32), _f32),
        jax.ShapeDtypeStruct((NP, 32), _f32),
    )
    scratch = [
        pltpu.VMEM((NP,), _f32),
        pltpu.VMEM((NP,), _f32),
        pltpu.VMEM((nbk, K), jnp.int32),
        pltpu.VMEM((nbk, K), jnp.int32),
        pltpu.VMEM((nbk, K), jnp.int32),
        pltpu.VMEM((nbk, K), jnp.int32),
        pltpu.VMEM((nbk, K), _f32),
        pltpu.VMEM((nbk, K), _f32),
        pltpu.VMEM((K, ETE), _f32),
        pltpu.VMEM((K, ETE), _f32),
        pltpu.VMEM((K, w), _f32),
        pltpu.VMEM((K, w), _f32),
        pltpu.VMEM((K, 32), _f32),
        pltpu.VMEM((K, 32), _f32),
        pltpu.VMEM((16,), _f32),
        pltpu.VMEM_SHARED((NP, w), _f32),
        pltpu.VMEM_SHARED((NP, 32), _f32),
        pltpu.SemaphoreType.DMA,
        pltpu.SemaphoreType.DMA,
        pltpu.SemaphoreType.DMA,
        pltpu.SemaphoreType.DMA,
        pltpu.SemaphoreType.DMA,
        pltpu.SemaphoreType.DMA,
    ]

    def body(src_hbm, dst_hbm, se_hbm, eae_hbm, ai_hbm, aj_hbm,
             hxa_hbm, hxb_hbm, shift_hbm,
             gxa_out, gxb_out, g32a_out, g32b_out,
             ai_v, aj_v, srcb0_v, srcb1_v, dstb0_v, dstb1_v, eb0_v, eb1_v,
             eae0_v, eae1_v, rows0_v, rows1_v, r320_v, r321_v,
             shift_v, accw_sh, acc32_sh, sg0, sg1, ss0, ss1, sb0, sb1):
        c = lax.axis_index("c")
        sid = lax.axis_index("s")
        pltpu.sync_copy(ai_hbm, ai_v)
        pltpu.sync_copy(aj_hbm, aj_v)
        pltpu.sync_copy(shift_hbm, shift_v)

        sbufs = [(srcb0_v, dstb0_v, eb0_v, sb0),
                 (srcb1_v, dstb1_v, eb1_v, sb1)]
        bufs = [(rows0_v, r320_v, eae0_v, sg0, ss0),
                (rows1_v, r321_v, eae1_v, sg1, ss1)]
        z16 = jnp.zeros((16,), _f32)

        def zrow(rr, carry):
            for jc in range(w // 16):
                rows0_v[rr, pl.ds(jc * 16, 16)] = z16
            r320_v[rr, pl.ds(0, 16)] = z16
            r320_v[rr, pl.ds(16, 16)] = z16
            return carry

        lax.fori_loop(0, K, zrow, 0)
        stripe = NP // 16

        def zcp(k, carry):
            base = sid * stripe + k * K
            pltpu.sync_copy(rows0_v, accw_sh.at[pl.ds(base, K)])
            pltpu.sync_copy(r320_v, acc32_sh.at[pl.ds(base, K)])
            return carry

        lax.fori_loop(0, stripe // K, zcp, 0)
        plsc.subcore_barrier()

        shiftvec = shift_v[...]
        lanei = lax.iota(jnp.int32, 16)

        def fire_block(j, p):
            srcb, dstb, eb, sb = sbufs[p]
            sl = pl.ds(j * nbk, nbk)
            pltpu.async_copy(src_hbm.at[sid, sl], srcb, sb)
            pltpu.async_copy(dst_hbm.at[sid, sl], dstb, sb)
            pltpu.async_copy(se_hbm.at[sid, sl], eb, sb)

        def wait_block(j, p):
            srcb, dstb, eb, sb = sbufs[p]
            sl = pl.ds(j * nbk, nbk)
            pltpu.make_async_copy(src_hbm.at[sid, sl], srcb, sb).wait()
            pltpu.make_async_copy(dst_hbm.at[sid, sl], dstb, sb).wait()
            pltpu.make_async_copy(se_hbm.at[sid, sl], eb, sb).wait()

        def drain_scatter(b, dstb):
            rows_b, r32_b, eae_b, sg, ss = bufs[b]
            pltpu.make_async_copy(rows_b, accw_sh.at[dstb.at[0]],
                                  ss).wait()

            @pl.when(c == b)
            def _():
                pltpu.make_async_copy(r32_b, acc32_sh.at[dstb.at[0]],
                                      ss).wait()

        def process_block(j, p):
            srcb, dstb, eb, sb = sbufs[p]

            def epass(ii, carry):
                for g in range(K // 16):
                    sl = pl.ds(g * 16, 16)
                    zv = (plsc.load_gather(ai_v, [dstb[ii, sl]])
                          + plsc.load_gather(aj_v, [srcb[ii, sl]])
                          + eb[ii, sl])
                    zv = jnp.where(zv >= 0, zv, NEG * zv)
                    eb[ii, sl] = jnp.exp(zv - shiftvec)
                return carry

            lax.fori_loop(0, nbk, epass, 0)

            def sstep(s, carry):
                for b in (0, 1):
                    ii = 2 * s + b
                    i_glob = j * nbk + ii
                    rows_b, r32_b, eae_b, sg, ss = bufs[b]

                    @pl.when(i_glob >= 2)
                    def _():
                        drain_scatter(b, dstb)

                    @pl.when(c == 0)
                    def _():
                        pltpu.async_copy(hxa_hbm.at[srcb.at[ii]], rows_b,
                                         sg)

                    @pl.when(c == 1)
                    def _():
                        pltpu.async_copy(hxb_hbm.at[srcb.at[ii]], rows_b,
                                         sg)

                    @pl.when(c == b)
                    def _():
                        off = sid * EPC + i_glob * K
                        pltpu.async_copy(eae_hbm.at[pl.ds(off, K)], eae_b,
                                         sg)
                for b in (0, 1):
                    ii = 2 * s + b
                    i_glob = j * nbk + ii
                    rows_b, r32_b, eae_b, sg, ss = bufs[b]
                    pltpu.make_async_copy(hxa_hbm.at[srcb.at[ii]], rows_b,
                                          sg).wait()

                    @pl.when(c == b)
                    def _():
                        off = sid * EPC + i_glob * K
                        pltpu.make_async_copy(eae_hbm.at[pl.ds(off, K)],
                                              eae_b, sg).wait()

                    for g in range(K // 16):
                        e16 = eb[ii, pl.ds(g * 16, 16)]
                        for r in range(16):
                            row = g * 16 + r
                            spl = _splat(e16, r)
                            for jc in range(w // 16):
                                cs = pl.ds(jc * 16, 16)
                                rows_b[row, cs] = rows_b[row, cs] * spl

                    @pl.when(c == b)
                    def _():
                        for g in range(K // 16):
                            e16 = eb[ii, pl.ds(g * 16, 16)]
                            for r in range(16):
                                row = g * 16 + r
                                spl = _splat(e16, r)
                                r32_b[row, pl.ds(0, 16)] = \
                                    eae_b[row, pl.ds(0, 16)] * spl
                                r32_b[row, pl.ds(16, 16)] = \
                                    jnp.where(lanei == 0, spl, 0.0)

                    pltpu.async_copy(rows_b, accw_sh.at[dstb.at[ii]], ss,
                                     add=True)

                    @pl.when(c == b)
                    def _():
                        pltpu.async_copy(r32_b, acc32_sh.at[dstb.at[ii]],
                                         ss, add=True)
                return carry

            lax.fori_loop(0, nbk // 2, sstep, 0)

        fire_block(0, 0)

        def blockloop(j, carry):
            @pl.when(j % 2 == 0)
            def _():
                wait_block(j, 0)

                @pl.when(j + 1 < nblk)
                def _():
                    fire_block(j + 1, 1)

                process_block(j, 0)

            @pl.when(j % 2 == 1)
            def _():
                wait_block(j, 1)

                @pl.when(j + 1 < nblk)
                def _():
                    fire_block(j + 1, 0)

                process_block(j, 1)

            return carry

        lax.fori_loop(0, nblk, blockloop, 0)
        drain_scatter(0, dstb0_v)
        drain_scatter(1, dstb0_v)
        plsc.subcore_barrier()
        base = sid * stripe

        @pl.when(c == 0)
        def _():
            pltpu.sync_copy(accw_sh.at[pl.ds(base, stripe)],
                            gxa_out.at[pl.ds(base, stripe)])
            pltpu.sync_copy(acc32_sh.at[pl.ds(base, stripe)],
                            g32a_out.at[pl.ds(base, stripe)])

        @pl.when(c == 1)
        def _():
            pltpu.sync_copy(accw_sh.at[pl.ds(base, stripe)],
                            gxb_out.at[pl.ds(base, stripe)])
            pltpu.sync_copy(acc32_sh.at[pl.ds(base, stripe)],
                            g32b_out.at[pl.ds(base, stripe)])

    return pl.kernel(body, out_type=out_type, mesh=mesh,
                     scratch_types=scratch,
                     compiler_params=pltpu.CompilerParams(
                         needs_layout_passes=False,
                         use_tc_tiling_on_sc=False))


def _full(shape):
    return pl.BlockSpec(shape, lambda i: tuple(0 for _ in shape))


def kernel(x, edge_index, node_type, edge_type, edge_attr, params):
    src = edge_index[0]
    dst = edge_index[1]
    xp = jnp.pad(x, ((0, NP - N), (0, 0)))
    ntf = jnp.pad(node_type.astype(_f32), (0, NP - N)).reshape(NP, 1)
    ps = [params['conv%d' % (i + 1)] for i in range(5)]

    oh = jax.nn.one_hot(edge_type, 6, dtype=_f32)
    x4 = jnp.concatenate([edge_attr, oh, jnp.zeros((E, 10), _f32)],
                         axis=1).reshape(E // 4, 128)
    w4ea_l, w4a_l, w4ts_l = [], [], []
    for l in range(5):
        p = ps[l]
        d = DOUTS[l]
        aw = p['att_W'][:, 0]
        we_v = aw[2 * d:2 * d + ETE]
        wa_v = aw[2 * d + ETE:2 * d + 2 * ETE]
        emb = p['edge_type_emb']
        ts = jnp.where(emb >= 0, emb, NEG * emb) @ we_v
        w4ea = jnp.zeros((128, 64), _f32)
        w4a = jnp.zeros((64, 4), _f32)
        w4ts = jnp.zeros((128, 4), _f32)
        for m in range(4):
            w4ea = w4ea.at[32 * m:32 * m + 16,
                           16 * m:16 * m + 16].set(p['edge_attr_W'])
            w4a = w4a.at[16 * m:16 * m + 16, m].set(wa_v)
            w4ts = w4ts.at[32 * m + 16:32 * m + 22, m].set(ts)
        w4ea_l.append(w4ea)
        w4a_l.append(w4a)
        w4ts_l.append(w4ts)
    w4ea_all = jnp.stack(w4ea_l)
    w4a_all = jnp.stack(w4a_l)
    w4ts_all = jnp.stack(w4ts_l)

    EB4 = 1600
    eae_p, s_p, maxs = pl.pallas_call(
        _prep_body,
        grid=(E // 4 // EB4,),
        in_specs=[
            pl.BlockSpec((EB4, 128), lambda i: (i, 0)),
            _full((5, 128, 64)),
            _full((5, 64, 4)),
            _full((5, 128, 4)),
        ],
        out_specs=[
            pl.BlockSpec((5, EB4, 64), lambda i: (0, i, 0)),
            pl.BlockSpec((5, EB4, 4), lambda i: (0, i, 0)),
            _full((8, 128)),
        ],
        out_shape=[
            jax.ShapeDtypeStruct((5, E // 4, 64), _f32),
            jax.ShapeDtypeStruct((5, E // 4, 4), _f32),
            jax.ShapeDtypeStruct((8, 128), _f32),
        ],
    )(x4, w4ea_all, w4a_all, w4ts_all)
    eae_all = eae_p.reshape(5, E, 16)
    s_all = s_p.reshape(5, E)

    def node_weights(l):
        p = ps[l]
        d = DOUTS[l]
        return (p['hetero_W'], p['hetero_b'][:, None, :],
                p['att_W'][:d], p['att_W'][d:2 * d], p['lin_W'][:d])

    def node_specs(l):
        din, d = DINS[l], DOUTS[l]
        w = W_L[l]
        nh = 2 * NCALLS[l]
        ins = [_full((3, din, d)), _full((3, 1, d)), _full((d, 1)),
               _full((d, 1)), _full((d, d))]
        outs = ([pl.BlockSpec((NB, 1), lambda i: (i, 0))] * 2
                + [pl.BlockSpec((NB, w), lambda i: (i, 0))] * nh
                + [_full((8, 128))])
        oshape = ([jax.ShapeDtypeStruct((NP, 1), _f32)] * 2
                  + [jax.ShapeDtypeStruct((NP, w), _f32)] * nh
                  + [jax.ShapeDtypeStruct((8, 128), _f32)])
        return ins, outs, oshape

    ins0, outs0, oshape0 = node_specs(0)
    ai, aj, *hx_list, mab = pl.pallas_call(
        _head_body,
        grid=(NP // NB,),
        in_specs=[pl.BlockSpec((NB, DINS[0]), lambda i: (i, 0)),
                  pl.BlockSpec((NB, 1), lambda i: (i, 0))] + ins0,
        out_specs=outs0,
        out_shape=oshape0,
    )(xp, ntf, *node_weights(0))

    src3 = src.reshape(16, NCH, K)
    dst3 = dst.reshape(16, NCH, K)
    s4 = s_all.reshape(5, 16, NCH, K)

    h = None
    for l in range(5):
        d = DOUTS[l]
        w = W_L[l]
        nc = NCALLS[l]
        shift = jnp.maximum(mab[0, 0] + mab[1, 0] + maxs[l, 0], 0.0)
        shift_arr = jnp.full((16,), shift, _f32)
        edge_k = _make_edge_kernel(w)
        gx_list = []
        g32a = g32b = None
        for k in range(nc):
            gxa, gxb, g32ak, g32bk = edge_k(src3, dst3, s4[l], eae_all[l],
                                            ai.reshape(NP), aj.reshape(NP),
                                            hx_list[2 * k],
                                            hx_list[2 * k + 1],
                                            shift_arr)
            gx_list += [gxa, gxb]
            if k == 0:
                g32a, g32b = g32ak, g32bk
        p = ps[l]
        merge_w = (p['lin_W'][d:], p['lin_b'][None, :],
                   p['ln_g'][None, :], p['ln_b'][None, :])
        merge_specs = [_full((16, d)), _full((1, d)), _full((1, d)),
                       _full((1, d))]
        gspecs = ([pl.BlockSpec((NB, w), lambda i: (i, 0))] * (2 * nc)
                  + [pl.BlockSpec((NB, 32), lambda i: (i, 0))] * 2)
        if l < 4:
            insn, outsn, oshapen = node_specs(l + 1)
            ai, aj, *hx_list, mab = pl.pallas_call(
                _make_mid_body(2 * nc),
                grid=(NP // NB,),
                in_specs=gspecs
                + [pl.BlockSpec((NB, 1), lambda i: (i, 0))]
                + merge_specs + insn,
                out_specs=outsn,
                out_shape=oshapen,
            )(*gx_list, g32a, g32b, ntf, *merge_w, *node_weights(l + 1))
        else:
            h = pl.pallas_call(
                _make_tail_body(2 * nc),
                grid=(NP // NB,),
                in_specs=gspecs + merge_specs,
                out_specs=pl.BlockSpec((NB, d), lambda i: (i, 0)),
                out_shape=jax.ShapeDtypeStruct((NP, d), _f32),
            )(*gx_list, g32a, g32b, *merge_w)
    return h[:N]
